# baseline (speedup 1.0000x reference)
import functools

import jax
import jax.numpy as jnp
from jax import lax
from jax.experimental import pallas as pl
from jax.experimental.pallas import tpu as pltpu
from jax.experimental.pallas import tpu_sc as plsc

N = 10000
D = 128
E = 320000

NC = 2
NS = 16
NW = NC * NS
C = 80
NCHUNK = 129
EPW = C * NCHUNK
EPAD = NW * EPW
NPAD = 10240
RPS = NPAD // NS
DEGW = 16


def _vmesh():
    return plsc.VectorSubcoreMesh(core_axis_name="c", subcore_axis_name="s")


def _sc_degree(dst3, ones_rows, zeros_deg):
    @functools.partial(
        pl.kernel,
        out_type=jax.ShapeDtypeStruct((NC, NPAD, DEGW), jnp.float32),
        mesh=_vmesh(),
        scratch_types=[
            pltpu.VMEM((NCHUNK, C), jnp.int32),
            pltpu.VMEM((C, DEGW), jnp.float32),
            pltpu.VMEM_SHARED((NPAD, DEGW), jnp.float32),
        ],
    )
    def k(dst_hbm, ones_hbm, zeros_hbm, out_hbm, didx, ones_v, acc):
        c = lax.axis_index("c")
        s = lax.axis_index("s")
        wid = c * NS + s
        pltpu.sync_copy(ones_hbm, ones_v)
        pltpu.sync_copy(dst_hbm.at[wid], didx)
        pltpu.sync_copy(zeros_hbm, acc.at[pl.ds(s * RPS, RPS)])
        plsc.subcore_barrier()

        @pl.loop(0, NCHUNK)
        def _(i):
            pltpu.sync_copy(ones_v, acc.at[didx.at[i]], add=True)

        plsc.subcore_barrier()
        pltpu.sync_copy(acc.at[pl.ds(s * RPS, RPS)],
                        out_hbm.at[c, pl.ds(s * RPS, RPS)])

    return k(dst3, ones_rows, zeros_deg)


def _sc_propagate(h, src3, dst3, zeros_feat):
    @functools.partial(
        pl.kernel,
        out_type=jax.ShapeDtypeStruct((NC, NPAD, D), jnp.float32),
        mesh=_vmesh(),
        scratch_types=[
            pltpu.VMEM((EPW,), jnp.int32),
            pltpu.VMEM((1, C), jnp.int32),
            pltpu.VMEM((1, C), jnp.int32),
            pltpu.VMEM((1, C), jnp.int32),
            pltpu.VMEM((C, D), jnp.float32),
            pltpu.VMEM((C, D), jnp.float32),
            pltpu.VMEM((C, D), jnp.float32),
            pltpu.SemaphoreType.DMA,
            pltpu.SemaphoreType.DMA,
            pltpu.SemaphoreType.DMA,
            pltpu.SemaphoreType.DMA,
            pltpu.SemaphoreType.DMA,
            pltpu.SemaphoreType.DMA,
            pltpu.VMEM_SHARED((NPAD, D), jnp.float32),
        ],
    )
    def k(h_hbm, src_hbm, dst_hbm, zeros_hbm, out_hbm,
          sidx, dstage0, dstage1, dstage2, rows0, rows1, rows2,
          gs0, gs1, gs2, ss0, ss1, ss2, acc):
        c = lax.axis_index("c")
        s = lax.axis_index("s")
        wid = c * NS + s
        rows = [rows0, rows1, rows2]
        dstage = [dstage0, dstage1, dstage2]
        gs = [gs0, gs1, gs2]
        ss = [ss0, ss1, ss2]
        pltpu.sync_copy(src_hbm.at[wid], sidx)
        pltpu.sync_copy(zeros_hbm, acc.at[pl.ds(s * RPS, RPS)])
        plsc.subcore_barrier()

        def stage(i, b):
            pltpu.sync_copy(dst_hbm.at[pl.ds(wid * EPW + i * C, C)],
                            dstage[b].at[0])

        def gather(i, b):
            pltpu.async_copy(h_hbm.at[sidx.at[pl.ds(i * C, C)]], rows[b], gs[b])

        def gwait(b):
            pltpu.make_async_copy(h_hbm.at[pl.ds(0, C)], rows[b], gs[b]).wait()

        def scat(b):
            pltpu.async_copy(rows[b], acc.at[dstage[b].at[0]], ss[b], add=True)

        def swait(b):
            pltpu.make_async_copy(rows[b], acc.at[pl.ds(0, C)], ss[b]).wait()

        stage(0, 0)
        gather(0, 0)
        stage(1, 1)
        gather(1, 1)

        @pl.loop(0, NCHUNK, step=3)
        def _(i):
            for j in range(3):
                k_ = i + j
                bn = (j + 2) % 3

                @pl.when(k_ >= 1)
                def _():
                    swait(bn)

                @pl.when(k_ + 2 < NCHUNK)
                def _():
                    stage(k_ + 2, bn)
                    gather(k_ + 2, bn)

                gwait(j)
                scat(j)

        swait((NCHUNK - 1) % 3)
        plsc.subcore_barrier()
        pltpu.sync_copy(acc.at[pl.ds(s * RPS, RPS)],
                        out_hbm.at[c, pl.ds(s * RPS, RPS)])

    return k(h, src3, dst3, zeros_feat)


def _tc_prep(features, dp):
    def body(f_ref, dp_ref, hs_ref, hinit_ref):
        indeg = dp_ref[0, :, 0:1] + dp_ref[1, :, 0:1]
        rs = lax.rsqrt(indeg)
        f = f_ref[...]
        hs_ref[...] = f * rs
        hinit_ref[...] = f * (rs * rs)

    return pl.pallas_call(
        body,
        out_shape=(
            jax.ShapeDtypeStruct((N, D), jnp.float32),
            jax.ShapeDtypeStruct((N, D), jnp.float32),
        ),
    )(features, dp)


def _tc_mid(p, dp, hinit):
    def body(p_ref, dp_ref, hinit_ref, out_ref):
        indeg = dp_ref[0, :, 0:1] + dp_ref[1, :, 0:1]
        rs = lax.rsqrt(indeg)
        h1 = (p_ref[0] + p_ref[1]) * rs + hinit_ref[...]
        out_ref[...] = h1 * rs

    return pl.pallas_call(
        body,
        out_shape=jax.ShapeDtypeStruct((N, D), jnp.float32),
    )(p, dp, hinit)


def _tc_final(p, dp, hinit):
    def body(p_ref, dp_ref, hinit_ref, out_ref):
        indeg = dp_ref[0, :, 0:1] + dp_ref[1, :, 0:1]
        rs = lax.rsqrt(indeg)
        out_ref[...] = (p_ref[0] + p_ref[1]) * rs + hinit_ref[...]

    return pl.pallas_call(
        body,
        out_shape=jax.ShapeDtypeStruct((N, D), jnp.float32),
    )(p, dp, hinit)


@jax.jit
def kernel(features, edge_index):
    src = edge_index[0]
    dst = edge_index[1]
    npad_e = EPAD - E
    src3 = jnp.concatenate(
        [src, jnp.zeros((npad_e,), jnp.int32)]).reshape(NW, EPW)
    dst_pad = jnp.concatenate(
        [dst, N + (jnp.arange(npad_e, dtype=jnp.int32) % (NPAD - N))])
    dst3 = dst_pad.reshape(NW, NCHUNK, C)
    dst2 = dst_pad

    ones_rows = jnp.ones((C, DEGW), jnp.float32)
    zeros_deg = jnp.zeros((RPS, DEGW), jnp.float32)
    zeros_feat = jnp.zeros((RPS, D), jnp.float32)

    dp = _sc_degree(dst3, ones_rows, zeros_deg)[:, :N]
    hs0, hinit = _tc_prep(features, dp)
    p1 = _sc_propagate(hs0, src3, dst2, zeros_feat)[:, :N]
    hs1 = _tc_mid(p1, dp, hinit)
    p2 = _sc_propagate(hs1, src3, dst2, zeros_feat)[:, :N]
    return _tc_final(p2, dp, hinit)


# speedup vs baseline: 1.5585x; 1.5585x over previous
import functools

import jax
import jax.numpy as jnp
from jax import lax
from jax.experimental import pallas as pl
from jax.experimental.pallas import tpu as pltpu
from jax.experimental.pallas import tpu_sc as plsc

N = 10000
D = 128
E = 320000

NC = 2
NS = 16
NW = NC * NS
C = 96
NCHUNK = 106
EPW = C * NCHUNK
EPAD = NW * EPW
NPAD = 10240
RPS = NPAD // NS
DEGW = 16


def _vmesh():
    return plsc.VectorSubcoreMesh(core_axis_name="c", subcore_axis_name="s")


def _sc_degree(dst3, ones_rows, zeros_deg):
    @functools.partial(
        pl.kernel,
        out_type=jax.ShapeDtypeStruct((NC, NPAD, DEGW), jnp.float32),
        mesh=_vmesh(),
        scratch_types=[
            pltpu.VMEM((NCHUNK, C), jnp.int32),
            pltpu.VMEM((C, DEGW), jnp.float32),
            pltpu.VMEM_SHARED((NPAD, DEGW), jnp.float32),
        ],
    )
    def k(dst_hbm, ones_hbm, zeros_hbm, out_hbm, didx, ones_v, acc):
        c = lax.axis_index("c")
        s = lax.axis_index("s")
        wid = c * NS + s
        pltpu.sync_copy(ones_hbm, ones_v)
        pltpu.sync_copy(dst_hbm.at[wid], didx)
        pltpu.sync_copy(zeros_hbm, acc.at[pl.ds(s * RPS, RPS)])
        plsc.subcore_barrier()

        @pl.loop(0, NCHUNK)
        def _(i):
            pltpu.sync_copy(ones_v, acc.at[didx.at[i]], add=True)

        plsc.subcore_barrier()
        pltpu.sync_copy(acc.at[pl.ds(s * RPS, RPS)],
                        out_hbm.at[c, pl.ds(s * RPS, RPS)])

    return k(dst3, ones_rows, zeros_deg)


def _sc_propagate(h, src3, dst3, zeros_feat):
    @functools.partial(
        pl.kernel,
        out_type=jax.ShapeDtypeStruct((NC, NPAD, D), jnp.float32),
        mesh=_vmesh(),
        scratch_types=[
            pltpu.VMEM((EPW,), jnp.int32),
            pltpu.VMEM((NCHUNK, C), jnp.int32),
            pltpu.VMEM((C, D), jnp.float32),
            pltpu.VMEM((C, D), jnp.float32),
            pltpu.VMEM_SHARED((NPAD, D), jnp.float32),
            pltpu.SemaphoreType.DMA,
            pltpu.SemaphoreType.DMA,
        ],
    )
    def k(h_hbm, src_hbm, dst_hbm, zeros_hbm, out_hbm,
          sidx, didx, rows0, rows1, acc, sem0, sem1):
        c = lax.axis_index("c")
        s = lax.axis_index("s")
        wid = c * NS + s
        pltpu.sync_copy(src_hbm.at[wid], sidx)
        pltpu.sync_copy(dst_hbm.at[wid], didx)
        pltpu.sync_copy(zeros_hbm, acc.at[pl.ds(s * RPS, RPS)])
        plsc.subcore_barrier()

        def gather(i, buf, sem):
            pltpu.async_copy(h_hbm.at[sidx.at[pl.ds(i * C, C)]], buf, sem)

        def gwait(buf, sem):
            pltpu.make_async_copy(h_hbm.at[pl.ds(0, C)], buf, sem).wait()

        gather(0, rows0, sem0)

        @pl.loop(0, NCHUNK, step=2)
        def _(i):
            gather(i + 1, rows1, sem1)
            gwait(rows0, sem0)
            pltpu.sync_copy(rows0, acc.at[didx.at[i]], add=True)

            @pl.when(i + 2 < NCHUNK)
            def _():
                gather(i + 2, rows0, sem0)

            gwait(rows1, sem1)
            pltpu.sync_copy(rows1, acc.at[didx.at[i + 1]], add=True)

        plsc.subcore_barrier()
        pltpu.sync_copy(acc.at[pl.ds(s * RPS, RPS)],
                        out_hbm.at[c, pl.ds(s * RPS, RPS)])

    return k(h, src3, dst3, zeros_feat)


def _tc_prep(features, dp):
    def body(f_ref, dp_ref, hs_ref, hinit_ref):
        indeg = dp_ref[0, :N, 0:1] + dp_ref[1, :N, 0:1]
        rs = lax.rsqrt(indeg)
        f = f_ref[...]
        hs_ref[...] = f * rs
        hinit_ref[...] = f * (rs * rs)

    return pl.pallas_call(
        body,
        out_shape=(
            jax.ShapeDtypeStruct((N, D), jnp.float32),
            jax.ShapeDtypeStruct((N, D), jnp.float32),
        ),
    )(features, dp)


def _tc_mid(p, dp, hinit):
    def body(p_ref, dp_ref, hinit_ref, out_ref):
        indeg = dp_ref[0, :N, 0:1] + dp_ref[1, :N, 0:1]
        rs = lax.rsqrt(indeg)
        h1 = (p_ref[0, :N] + p_ref[1, :N]) * rs + hinit_ref[...]
        out_ref[...] = h1 * rs

    return pl.pallas_call(
        body,
        out_shape=jax.ShapeDtypeStruct((N, D), jnp.float32),
    )(p, dp, hinit)


def _tc_final(p, dp, hinit):
    def body(p_ref, dp_ref, hinit_ref, out_ref):
        indeg = dp_ref[0, :N, 0:1] + dp_ref[1, :N, 0:1]
        rs = lax.rsqrt(indeg)
        out_ref[...] = (p_ref[0, :N] + p_ref[1, :N]) * rs + hinit_ref[...]

    return pl.pallas_call(
        body,
        out_shape=jax.ShapeDtypeStruct((N, D), jnp.float32),
    )(p, dp, hinit)


@jax.jit
def kernel(features, edge_index):
    src = edge_index[0]
    dst = edge_index[1]
    npad_e = EPAD - E
    src3 = jnp.concatenate(
        [src, jnp.zeros((npad_e,), jnp.int32)]).reshape(NW, EPW)
    dst_pad = jnp.concatenate(
        [dst, N + (jnp.arange(npad_e, dtype=jnp.int32) % (NPAD - N))])
    dst3 = dst_pad.reshape(NW, NCHUNK, C)

    ones_rows = jnp.ones((C, DEGW), jnp.float32)
    zeros_deg = jnp.zeros((RPS, DEGW), jnp.float32)
    zeros_feat = jnp.zeros((RPS, D), jnp.float32)

    dp = _sc_degree(dst3, ones_rows, zeros_deg)
    hs0, hinit = _tc_prep(features, dp)
    p1 = _sc_propagate(hs0, src3, dst3, zeros_feat)
    hs1 = _tc_mid(p1, dp, hinit)
    p2 = _sc_propagate(hs1, src3, dst3, zeros_feat)
    return _tc_final(p2, dp, hinit)


# speedup vs baseline: 3.8749x; 2.4863x over previous
import functools

import jax
import jax.numpy as jnp
from jax import lax
from jax.experimental import pallas as pl
from jax.experimental.pallas import tpu as pltpu
from jax.experimental.pallas import tpu_sc as plsc

N = 10000
D = 128
E = 320000

NC = 2
NS = 16
NW = NC * NS
C = 96
NCHUNK = 106
EPW = C * NCHUNK
EPAD = NW * EPW
NPAD = 10240
RPS = NPAD // NS
DEGW = 16


def _vmesh():
    return plsc.VectorSubcoreMesh(core_axis_name="c", subcore_axis_name="s")


def _sc_degree(dst3, ones_rows, zeros_deg):
    @functools.partial(
        pl.kernel,
        out_type=jax.ShapeDtypeStruct((NC, NPAD, DEGW), jnp.float32),
        mesh=_vmesh(),
        scratch_types=[
            pltpu.VMEM((NCHUNK, C), jnp.int32),
            pltpu.VMEM((C, DEGW), jnp.float32),
            pltpu.VMEM_SHARED((NPAD, DEGW), jnp.float32),
        ],
    )
    def k(dst_hbm, ones_hbm, zeros_hbm, out_hbm, didx, ones_v, acc):
        c = lax.axis_index("c")
        s = lax.axis_index("s")
        wid = c * NS + s
        pltpu.sync_copy(ones_hbm, ones_v)
        pltpu.sync_copy(dst_hbm.at[wid], didx)
        pltpu.sync_copy(zeros_hbm, acc.at[pl.ds(s * RPS, RPS)])
        plsc.subcore_barrier()

        @pl.loop(0, NCHUNK)
        def _(i):
            pltpu.sync_copy(ones_v, acc.at[didx.at[i]], add=True)

        plsc.subcore_barrier()
        pltpu.sync_copy(acc.at[pl.ds(s * RPS, RPS)],
                        out_hbm.at[c, pl.ds(s * RPS, RPS)])

    return k(dst3, ones_rows, zeros_deg)


def _sc_propagate(h, src3, dst3, zeros_feat):
    @functools.partial(
        pl.kernel,
        out_type=jax.ShapeDtypeStruct((NC, NPAD, D), jnp.float32),
        mesh=_vmesh(),
        scratch_types=[
            pltpu.VMEM((EPW,), jnp.int32),
            pltpu.VMEM((NCHUNK, C), jnp.int32),
            pltpu.VMEM((C, D), jnp.float32),
            pltpu.VMEM((C, D), jnp.float32),
            pltpu.VMEM_SHARED((NPAD, D), jnp.float32),
            pltpu.SemaphoreType.DMA,
            pltpu.SemaphoreType.DMA,
        ],
    )
    def k(h_hbm, src_hbm, dst_hbm, zeros_hbm, out_hbm,
          sidx, didx, rows0, rows1, acc, sem0, sem1):
        c = lax.axis_index("c")
        s = lax.axis_index("s")
        wid = c * NS + s
        pltpu.sync_copy(src_hbm.at[wid], sidx)
        pltpu.sync_copy(dst_hbm.at[wid], didx)
        pltpu.sync_copy(zeros_hbm, acc.at[pl.ds(s * RPS, RPS)])
        plsc.subcore_barrier()

        def gather(i, buf, sem):
            pltpu.async_copy(h_hbm.at[sidx.at[pl.ds(i * C, C)]], buf, sem)

        def gwait(buf, sem):
            pltpu.make_async_copy(h_hbm.at[pl.ds(0, C)], buf, sem).wait()

        gather(0, rows0, sem0)

        @pl.loop(0, NCHUNK, step=2)
        def _(i):
            gather(i + 1, rows1, sem1)
            gwait(rows0, sem0)
            pltpu.sync_copy(rows0, acc.at[didx.at[i]], add=True)

            @pl.when(i + 2 < NCHUNK)
            def _():
                gather(i + 2, rows0, sem0)

            gwait(rows1, sem1)
            pltpu.sync_copy(rows1, acc.at[didx.at[i + 1]], add=True)

        plsc.subcore_barrier()
        pltpu.sync_copy(acc.at[pl.ds(s * RPS, RPS)],
                        out_hbm.at[c, pl.ds(s * RPS, RPS)])

    return k(h, src3, dst3, zeros_feat)


def _tc_prep(features, dp):
    def body(f_ref, dp_ref, hs_ref, hinit_ref):
        indeg = dp_ref[0, :N, 0:1] + dp_ref[1, :N, 0:1]
        rs = lax.rsqrt(indeg)
        f = f_ref[...]
        hs_ref[:N] = f * rs
        hs_ref[N:] = jnp.zeros((NPAD - N, D), jnp.float32)
        hinit_ref[...] = f * (rs * rs)

    return pl.pallas_call(
        body,
        out_shape=(
            jax.ShapeDtypeStruct((NPAD, D), jnp.float32),
            jax.ShapeDtypeStruct((N, D), jnp.float32),
        ),
    )(features, dp)


def _tc_mid(p, dp, hinit):
    def body(p_ref, dp_ref, hinit_ref, out_ref):
        indeg = dp_ref[0, :N, 0:1] + dp_ref[1, :N, 0:1]
        rs = lax.rsqrt(indeg)
        h1 = (p_ref[0, :N] + p_ref[1, :N]) * rs + hinit_ref[...]
        out_ref[:N] = h1 * rs
        out_ref[N:] = jnp.zeros((NPAD - N, D), jnp.float32)

    return pl.pallas_call(
        body,
        out_shape=jax.ShapeDtypeStruct((NPAD, D), jnp.float32),
    )(p, dp, hinit)


def _tc_final(p, dp, hinit):
    def body(p_ref, dp_ref, hinit_ref, out_ref):
        indeg = dp_ref[0, :N, 0:1] + dp_ref[1, :N, 0:1]
        rs = lax.rsqrt(indeg)
        out_ref[...] = (p_ref[0, :N] + p_ref[1, :N]) * rs + hinit_ref[...]

    return pl.pallas_call(
        body,
        out_shape=jax.ShapeDtypeStruct((N, D), jnp.float32),
    )(p, dp, hinit)


@jax.jit
def kernel(features, edge_index):
    src = edge_index[0]
    dst = edge_index[1]
    npad_e = EPAD - E
    iota_e = jnp.arange(npad_e, dtype=jnp.int32)
    src3 = jnp.concatenate(
        [src, N + iota_e % (NPAD - N)]).reshape(NW, EPW)
    dst3 = jnp.concatenate(
        [dst, iota_e % NPAD]).reshape(NW, NCHUNK, C)
    dst3deg = jnp.concatenate(
        [dst, N + iota_e % (NPAD - N)]).reshape(NW, NCHUNK, C)

    ones_rows = jnp.ones((C, DEGW), jnp.float32)
    zeros_deg = jnp.zeros((RPS, DEGW), jnp.float32)
    zeros_feat = jnp.zeros((RPS, D), jnp.float32)

    dp = _sc_degree(dst3deg, ones_rows, zeros_deg)
    hs0, hinit = _tc_prep(features, dp)
    p1 = _sc_propagate(hs0, src3, dst3, zeros_feat)
    hs1 = _tc_mid(p1, dp, hinit)
    p2 = _sc_propagate(hs1, src3, dst3, zeros_feat)
    return _tc_final(p2, dp, hinit)


# speedup vs baseline: 4.2922x; 1.1077x over previous
import functools

import jax
import jax.numpy as jnp
from jax import lax
from jax.experimental import pallas as pl
from jax.experimental.pallas import tpu as pltpu
from jax.experimental.pallas import tpu_sc as plsc

N = 10000
D = 128
E = 320000

NC = 2
NS = 16
NW = NC * NS
C = 80
NCHUNK = 129
EPW = C * NCHUNK
EPAD = NW * EPW
CD = 96
NCHUNKD = 106
EPWD = CD * NCHUNKD
EPADD = NW * EPWD
NPAD = 10240
RPS = NPAD // NS
DEGW = 16


def _vmesh():
    return plsc.VectorSubcoreMesh(core_axis_name="c", subcore_axis_name="s")


def _sc_degree(dst3, ones_rows, zeros_deg):
    @functools.partial(
        pl.kernel,
        out_type=jax.ShapeDtypeStruct((NC, NPAD, DEGW), jnp.float32),
        mesh=_vmesh(),
        scratch_types=[
            pltpu.VMEM((NCHUNKD, CD), jnp.int32),
            pltpu.VMEM((CD, DEGW), jnp.float32),
            pltpu.SemaphoreType.DMA,
            pltpu.VMEM_SHARED((NPAD, DEGW), jnp.float32),
        ],
    )
    def k(dst_hbm, ones_hbm, zeros_hbm, out_hbm, didx, ones_v, ssem, acc):
        c = lax.axis_index("c")
        s = lax.axis_index("s")
        wid = c * NS + s
        pltpu.sync_copy(ones_hbm, ones_v)
        pltpu.sync_copy(dst_hbm.at[wid], didx)
        pltpu.sync_copy(zeros_hbm, acc.at[pl.ds(s * RPS, RPS)])
        plsc.subcore_barrier()

        @pl.loop(0, NCHUNKD)
        def _(i):
            pltpu.sync_copy(ones_v, acc.at[didx.at[i]], add=True)

        plsc.subcore_barrier()
        pltpu.sync_copy(acc.at[pl.ds(s * RPS, RPS)],
                        out_hbm.at[c, pl.ds(s * RPS, RPS)])

    return k(dst3, ones_rows, zeros_deg)


def _sc_propagate(h, src3, dst3, zeros_feat):
    @functools.partial(
        pl.kernel,
        out_type=jax.ShapeDtypeStruct((NC, NPAD, D), jnp.float32),
        mesh=_vmesh(),
        scratch_types=[
            pltpu.VMEM((EPW,), jnp.int32),
            pltpu.VMEM((1, C), jnp.int32),
            pltpu.VMEM((1, C), jnp.int32),
            pltpu.VMEM((1, C), jnp.int32),
            pltpu.VMEM((C, D), jnp.float32),
            pltpu.VMEM((C, D), jnp.float32),
            pltpu.VMEM((C, D), jnp.float32),
            pltpu.SemaphoreType.DMA,
            pltpu.SemaphoreType.DMA,
            pltpu.SemaphoreType.DMA,
            pltpu.SemaphoreType.DMA,
            pltpu.SemaphoreType.DMA,
            pltpu.SemaphoreType.DMA,
            pltpu.SemaphoreType.DMA,
            pltpu.SemaphoreType.DMA,
            pltpu.SemaphoreType.DMA,
            pltpu.VMEM_SHARED((NPAD, D), jnp.float32),
        ],
    )
    def k(h_hbm, src_hbm, dst_hbm, zeros_hbm, out_hbm,
          sidx, dg0, dg1, dg2, r0, r1, r2,
          gs0, gs1, gs2, ss0, ss1, ss2, ds0, ds1, ds2, acc):
        c = lax.axis_index("c")
        s = lax.axis_index("s")
        wid = c * NS + s
        rows = [r0, r1, r2]
        dstage = [dg0, dg1, dg2]
        gs = [gs0, gs1, gs2]
        ss = [ss0, ss1, ss2]
        ds = [ds0, ds1, ds2]
        pltpu.sync_copy(src_hbm.at[wid], sidx)
        pltpu.sync_copy(zeros_hbm, acc.at[pl.ds(s * RPS, RPS)])
        plsc.subcore_barrier()
        base = wid * EPW

        def stage(i, b):
            pltpu.async_copy(dst_hbm.at[pl.ds(base + i * C, C)],
                             dstage[b].at[0], ds[b])

        def dwait(b):
            pltpu.make_async_copy(dst_hbm.at[pl.ds(0, C)],
                                  dstage[b].at[0], ds[b]).wait()

        def gather(i, b):
            pltpu.async_copy(h_hbm.at[sidx.at[pl.ds(i * C, C)]], rows[b], gs[b])

        def gwait(b):
            pltpu.make_async_copy(h_hbm.at[pl.ds(0, C)], rows[b], gs[b]).wait()

        def scat(b):
            pltpu.async_copy(rows[b], acc.at[dstage[b].at[0]], ss[b], add=True)

        def swait(b):
            pltpu.make_async_copy(rows[b], acc.at[pl.ds(0, C)], ss[b]).wait()

        stage(0, 0)
        gather(0, 0)
        stage(1, 1)
        gather(1, 1)

        @pl.loop(0, NCHUNK, step=3)
        def _(i):
            for j in range(3):
                k_ = i + j
                bn = (j + 2) % 3

                @pl.when(k_ >= 1)
                def _():
                    swait(bn)

                @pl.when(k_ + 2 < NCHUNK)
                def _():
                    stage(k_ + 2, bn)
                    gather(k_ + 2, bn)

                gwait(j)
                dwait(j)
                scat(j)

        swait((NCHUNK - 1) % 3)
        plsc.subcore_barrier()
        pltpu.sync_copy(acc.at[pl.ds(s * RPS, RPS)],
                        out_hbm.at[c, pl.ds(s * RPS, RPS)])

    return k(h, src3, dst3, zeros_feat)


def _tc_prep(features, dp):
    def body(f_ref, dp_ref, hs_ref, hinit_ref):
        indeg = dp_ref[0, :N, 0:1] + dp_ref[1, :N, 0:1]
        rs = lax.rsqrt(indeg)
        f = f_ref[...]
        hs_ref[:N] = f * rs
        hs_ref[N:] = jnp.zeros((NPAD - N, D), jnp.float32)
        hinit_ref[...] = f * (rs * rs)

    return pl.pallas_call(
        body,
        out_shape=(
            jax.ShapeDtypeStruct((NPAD, D), jnp.float32),
            jax.ShapeDtypeStruct((N, D), jnp.float32),
        ),
    )(features, dp)


def _tc_mid(p, dp, hinit):
    def body(p_ref, dp_ref, hinit_ref, out_ref):
        indeg = dp_ref[0, :N, 0:1] + dp_ref[1, :N, 0:1]
        rs = lax.rsqrt(indeg)
        h1 = (p_ref[0, :N] + p_ref[1, :N]) * rs + hinit_ref[...]
        out_ref[:N] = h1 * rs
        out_ref[N:] = jnp.zeros((NPAD - N, D), jnp.float32)

    return pl.pallas_call(
        body,
        out_shape=jax.ShapeDtypeStruct((NPAD, D), jnp.float32),
    )(p, dp, hinit)


def _tc_final(p, dp, hinit):
    def body(p_ref, dp_ref, hinit_ref, out_ref):
        indeg = dp_ref[0, :N, 0:1] + dp_ref[1, :N, 0:1]
        rs = lax.rsqrt(indeg)
        out_ref[...] = (p_ref[0, :N] + p_ref[1, :N]) * rs + hinit_ref[...]

    return pl.pallas_call(
        body,
        out_shape=jax.ShapeDtypeStruct((N, D), jnp.float32),
    )(p, dp, hinit)


@jax.jit
def kernel(features, edge_index):
    src = edge_index[0]
    dst = edge_index[1]
    iota_p = jnp.arange(EPAD - E, dtype=jnp.int32)
    src3 = jnp.concatenate(
        [src, N + iota_p % (NPAD - N)]).reshape(NW, EPW)
    dst2 = jnp.concatenate([dst, iota_p % NPAD])
    iota_d = jnp.arange(EPADD - E, dtype=jnp.int32)
    dst3deg = jnp.concatenate(
        [dst, N + iota_d % (NPAD - N)]).reshape(NW, NCHUNKD, CD)

    ones_rows = jnp.ones((CD, DEGW), jnp.float32)
    zeros_deg = jnp.zeros((RPS, DEGW), jnp.float32)
    zeros_feat = jnp.zeros((RPS, D), jnp.float32)

    dp = _sc_degree(dst3deg, ones_rows, zeros_deg)
    hs0, hinit = _tc_prep(features, dp)
    p1 = _sc_propagate(hs0, src3, dst2, zeros_feat)
    hs1 = _tc_mid(p1, dp, hinit)
    p2 = _sc_propagate(hs1, src3, dst2, zeros_feat)
    return _tc_final(p2, dp, hinit)
